# baseline (speedup 1.0000x reference)
import jax
import jax.numpy as jnp
from jax.experimental import pallas as pl
from jax.experimental.pallas import tpu as pltpu
from jax.experimental.pallas import tpu_sc as plsc

NC = 4
NL = 3
FPC = 32
GH = 1024
GE = 512
LD = 128
K = 2048
IN = 128

BT = 512
DT = 1024
GW = 128

_PREC = jax.lax.Precision.DEFAULT


def _cn_body(cbt_ref, cn_ref):
    cbt = cbt_ref[0]
    cn_ref[0] = jnp.sum(cbt * cbt, axis=0, keepdims=True)


def _vq_codes_body(x_ref, w1_ref, b1_ref, w2_ref, b2_ref, pw_ref, pb_ref,
                   cbt_ref, cn_ref, out_ref):
    x = x_ref[...]
    h_all = jnp.maximum(
        jnp.dot(x, w1_ref[...], preferred_element_type=jnp.float32,
                precision=_PREC) + b1_ref[...], 0.0)
    for c in range(NC):
        h = h_all[:, c * GH:(c + 1) * GH]
        emb = jnp.dot(h, w2_ref[c], preferred_element_type=jnp.float32,
                      precision=_PREC) + b2_ref[c][None, :]
        zc = jnp.dot(emb, pw_ref[c], preferred_element_type=jnp.float32,
                     precision=_PREC) + pb_ref[c][None, :]
        for l in range(NL):
            j = c * NL + l
            z2 = zc[:, l * LD:(l + 1) * LD]
            cbt = cbt_ref[j]
            scores2 = jnp.dot(z2, cbt, preferred_element_type=jnp.float32,
                              precision=_PREC)
            d = scores2 + cn_ref[j]
            m = jnp.min(d, axis=1, keepdims=True)
            iota_f = jax.lax.broadcasted_iota(
                jnp.int32, d.shape, 1).astype(jnp.float32)
            code_f = jnp.min(jnp.where(d <= m, iota_f, float(K)), axis=1)
            out_ref[:, j] = code_f.astype(jnp.int32) + j * K
    for j in range(NC * NL, 16):
        out_ref[:, j] = jnp.zeros((BT,), jnp.int32)


def _decoder_body(zq_ref, w1_ref, b1_ref, w2_ref, b2_ref, out_ref):
    acc = jnp.dot(zq_ref[0], w1_ref[0], preferred_element_type=jnp.float32,
                  precision=_PREC)
    for j in range(1, NC * NL):
        acc = acc + jnp.dot(zq_ref[j], w1_ref[j],
                            preferred_element_type=jnp.float32,
                            precision=_PREC)
    h = jnp.maximum(acc + b1_ref[...], 0.0)
    out_ref[...] = jnp.dot(h, w2_ref[...], preferred_element_type=jnp.float32,
                           precision=_PREC) + b2_ref[...]


_NW = 32
_NBUF = 4


def _sc_gather(cb_flat, idx):
    num = idx.shape[0]
    width = cb_flat.shape[1]
    per_w = num // _NW
    nblk = per_w // GW
    mesh = plsc.VectorSubcoreMesh(core_axis_name="c", subcore_axis_name="s")

    @pl.kernel(out_type=jax.ShapeDtypeStruct((num, width), cb_flat.dtype),
               mesh=mesh,
               scratch_types=([pltpu.VMEM((per_w,), jnp.int32)]
                              + [pltpu.VMEM((GW, width), cb_flat.dtype)
                                 for _ in range(_NBUF)]
                              + [pltpu.SemaphoreType.DMA] * (2 * _NBUF)))
    def gather_kernel(cb_hbm, i_hbm, o_hbm, idx_v, *bufs_and_sems):
        rows = bufs_and_sems[:_NBUF]
        sg = bufs_and_sems[_NBUF:2 * _NBUF]
        so = bufs_and_sems[2 * _NBUF:]
        wid = jax.lax.axis_index("c") * 16 + jax.lax.axis_index("s")
        base = wid * per_w
        pltpu.sync_copy(i_hbm.at[pl.ds(base, per_w)], idx_v)
        gath = [None] * _NBUF
        outc = [None] * _NBUF
        for k in range(nblk):
            b = k % _NBUF
            if k >= _NBUF:
                outc[b].wait()
            gath[b] = pltpu.async_copy(
                cb_hbm.at[idx_v.at[pl.ds(k * GW, GW)]], rows[b], sg[b])
            if k >= 2:
                p = (k - 2) % _NBUF
                gath[p].wait()
                outc[p] = pltpu.async_copy(
                    rows[p], o_hbm.at[pl.ds(base + (k - 2) * GW, GW)], so[p])
        for k in (nblk - 2, nblk - 1):
            p = k % _NBUF
            gath[p].wait()
            outc[p] = pltpu.async_copy(
                rows[p], o_hbm.at[pl.ds(base + k * GW, GW)], so[p])
        for k in range(max(0, nblk - _NBUF), nblk):
            outc[k % _NBUF].wait()

    return gather_kernel(cb_flat, idx)


def _cn_call(cbt):
    return pl.pallas_call(
        _cn_body,
        grid=(NC * NL,),
        in_specs=[pl.BlockSpec((1, LD, K), lambda j: (j, 0, 0))],
        out_specs=pl.BlockSpec((1, 1, K), lambda j: (j, 0, 0)),
        out_shape=jax.ShapeDtypeStruct((NC * NL, 1, K), jnp.float32),
    )(cbt)


def _codes_call(x_t, w1_bd, b1_all, fe_w2, fe_b2, pw_cat, pb_cat, cbt, cn):
    Bc = x_t.shape[0]
    return pl.pallas_call(
        _vq_codes_body,
        grid=(Bc // BT,),
        in_specs=[
            pl.BlockSpec((BT, IN), lambda i: (i, 0)),
            pl.BlockSpec((IN, NC * GH), lambda i: (0, 0)),
            pl.BlockSpec((1, NC * GH), lambda i: (0, 0)),
            pl.BlockSpec((NC, GH, GE), lambda i: (0, 0, 0)),
            pl.BlockSpec((NC, GE), lambda i: (0, 0)),
            pl.BlockSpec((NC, GE, NL * LD), lambda i: (0, 0, 0)),
            pl.BlockSpec((NC, NL * LD), lambda i: (0, 0)),
            pl.BlockSpec((NC * NL, LD, K), lambda i: (0, 0, 0)),
            pl.BlockSpec((NC * NL, 1, K), lambda i: (0, 0, 0)),
        ],
        out_specs=pl.BlockSpec((BT, 16), lambda i: (i, 0)),
        out_shape=jax.ShapeDtypeStruct((Bc, 16), jnp.int32),
    )(x_t, w1_bd, b1_all, fe_w2, fe_b2, pw_cat, pb_cat, cbt, cn)


def _decoder_call(zq3, dec_w1, dec_b1, dec_w2, dec_b2):
    Bc = zq3.shape[1]
    return pl.pallas_call(
        _decoder_body,
        grid=(Bc // DT,),
        in_specs=[
            pl.BlockSpec((NC * NL, DT, LD), lambda i: (0, i, 0)),
            pl.BlockSpec((NC * NL, LD, 256), lambda i: (0, 0, 0)),
            pl.BlockSpec((1, 256), lambda i: (0, 0)),
            pl.BlockSpec((256, IN), lambda i: (0, 0)),
            pl.BlockSpec((1, IN), lambda i: (0, 0)),
        ],
        out_specs=pl.BlockSpec((DT, IN), lambda i: (i, 0)),
        out_shape=jax.ShapeDtypeStruct((Bc, IN), jnp.float32),
    )(zq3, dec_w1, dec_b1, dec_w2, dec_b2)


NCHUNK = 2


def kernel(x, fe_w1, fe_b1, fe_w2, fe_b2, proj_w, proj_b, codebooks,
           dec_w1, dec_b1, dec_w2, dec_b2):
    B = x.shape[0]

    w1_bd = jnp.zeros((IN, NC * GH), jnp.float32)
    for c in range(NC):
        w1_bd = w1_bd.at[c * FPC:(c + 1) * FPC, c * GH:(c + 1) * GH].set(fe_w1[c])
    b1_all = fe_b1.reshape(1, NC * GH)
    pw_cat = -2.0 * jnp.transpose(proj_w.reshape(NC, NL, GE, LD),
                                  (0, 2, 1, 3)).reshape(NC, GE, NL * LD)
    pb_cat = -2.0 * proj_b.reshape(NC, NL * LD)
    cbt = jnp.transpose(codebooks, (0, 2, 1))
    cn = _cn_call(cbt)
    w1_bd = w1_bd.astype(jnp.bfloat16)
    fe_w2 = fe_w2.astype(jnp.bfloat16)
    pw_cat = pw_cat.astype(jnp.bfloat16)
    cbt = cbt.astype(jnp.bfloat16)
    cb_flat = codebooks.reshape(NC * NL * K, LD)
    db1 = dec_b1.reshape(1, 256)
    db2 = dec_b2.reshape(1, IN)
    dw1 = dec_w1.reshape(NC * NL, LD, 256).astype(jnp.bfloat16)

    sizes = [B * 9 // 16, B * 7 // 16] if NCHUNK == 2 else [B // NCHUNK] * NCHUNK
    recons = []
    off = 0
    for t in range(NCHUNK):
        Bc = sizes[t]
        x_t = jax.lax.slice_in_dim(x, off, off + Bc, axis=0)
        off += Bc
        codes = _codes_call(x_t, w1_bd, b1_all, fe_w2, fe_b2, pw_cat,
                            pb_cat, cbt, cn)
        idx = codes[:, :NC * NL].T.reshape(Bc * NC * NL)
        zq3 = _sc_gather(cb_flat, idx).reshape(NC * NL, Bc, LD)
        recons.append(_decoder_call(zq3, dw1, db1, dec_w2, db2))
    return jnp.concatenate(recons, axis=0)


# speedup vs baseline: 1.0429x; 1.0429x over previous
import jax
import jax.numpy as jnp
from jax.experimental import pallas as pl
from jax.experimental.pallas import tpu as pltpu
from jax.experimental.pallas import tpu_sc as plsc

NC = 4
NL = 3
FPC = 32
GH = 1024
GE = 512
LD = 128
K = 2048
IN = 128

BT = 512
DT = 1024
GW = 128

_PREC = jax.lax.Precision.DEFAULT


def _cn_body(cbt_ref, cn_ref):
    cbt = cbt_ref[0]
    cn_ref[0] = jnp.sum(cbt * cbt, axis=0, keepdims=True)


def _vq_codes_body(x_ref, w1_ref, b1_ref, w2_ref, b2_ref, pw_ref, pb_ref,
                   cbt_ref, cn_ref, out_ref):
    x = x_ref[...]
    h_all = jnp.maximum(
        jnp.dot(x, w1_ref[...], preferred_element_type=jnp.float32,
                precision=_PREC) + b1_ref[...], 0.0)
    for c in range(NC):
        h = h_all[:, c * GH:(c + 1) * GH]
        emb = jnp.dot(h, w2_ref[c], preferred_element_type=jnp.float32,
                      precision=_PREC) + b2_ref[c][None, :]
        zc = jnp.dot(emb, pw_ref[c], preferred_element_type=jnp.float32,
                     precision=_PREC) + pb_ref[c][None, :]
        for l in range(NL):
            j = c * NL + l
            z2 = zc[:, l * LD:(l + 1) * LD]
            cbt = cbt_ref[j]
            scores2 = jnp.dot(z2, cbt, preferred_element_type=jnp.float32,
                              precision=_PREC)
            d = scores2 + cn_ref[j]
            m = jnp.min(d, axis=1, keepdims=True)
            iota_f = jax.lax.broadcasted_iota(
                jnp.int32, d.shape, 1).astype(jnp.float32)
            code_f = jnp.min(jnp.where(d <= m, iota_f, float(K)), axis=1)
            out_ref[:, j] = code_f.astype(jnp.int32) + j * K
    for j in range(NC * NL, 16):
        out_ref[:, j] = jnp.zeros((BT,), jnp.int32)


def _decoder_body(zq_ref, w1_ref, b1_ref, w2_ref, b2_ref, out_ref):
    acc = jnp.dot(zq_ref[0], w1_ref[0], preferred_element_type=jnp.float32,
                  precision=_PREC)
    for j in range(1, NC * NL):
        acc = acc + jnp.dot(zq_ref[j], w1_ref[j],
                            preferred_element_type=jnp.float32,
                            precision=_PREC)
    h = jnp.maximum(acc + b1_ref[...], 0.0)
    out_ref[...] = jnp.dot(h, w2_ref[...], preferred_element_type=jnp.float32,
                           precision=_PREC) + b2_ref[...]


_NW = 32
_NBUF = 4


def _sc_gather(cb_flat, idx):
    num = idx.shape[0]
    width = cb_flat.shape[1]
    per_w = num // _NW
    nblk = per_w // GW
    mesh = plsc.VectorSubcoreMesh(core_axis_name="c", subcore_axis_name="s")

    @pl.kernel(out_type=jax.ShapeDtypeStruct((num, width), cb_flat.dtype),
               mesh=mesh,
               scratch_types=([pltpu.VMEM((per_w,), jnp.int32)]
                              + [pltpu.VMEM((GW, width), cb_flat.dtype)
                                 for _ in range(_NBUF)]
                              + [pltpu.SemaphoreType.DMA] * (2 * _NBUF)))
    def gather_kernel(cb_hbm, i_hbm, o_hbm, idx_v, *bufs_and_sems):
        rows = bufs_and_sems[:_NBUF]
        sg = bufs_and_sems[_NBUF:2 * _NBUF]
        so = bufs_and_sems[2 * _NBUF:]
        wid = jax.lax.axis_index("c") * 16 + jax.lax.axis_index("s")
        base = wid * per_w
        pltpu.sync_copy(i_hbm.at[pl.ds(base, per_w)], idx_v)
        gath = [None] * _NBUF
        outc = [None] * _NBUF
        for k in range(nblk):
            b = k % _NBUF
            if k >= _NBUF:
                outc[b].wait()
            gath[b] = pltpu.async_copy(
                cb_hbm.at[idx_v.at[pl.ds(k * GW, GW)]], rows[b], sg[b])
            if k >= 2:
                p = (k - 2) % _NBUF
                gath[p].wait()
                outc[p] = pltpu.async_copy(
                    rows[p], o_hbm.at[pl.ds(base + (k - 2) * GW, GW)], so[p])
        for k in (nblk - 2, nblk - 1):
            p = k % _NBUF
            gath[p].wait()
            outc[p] = pltpu.async_copy(
                rows[p], o_hbm.at[pl.ds(base + k * GW, GW)], so[p])
        for k in range(max(0, nblk - _NBUF), nblk):
            outc[k % _NBUF].wait()

    return gather_kernel(cb_flat, idx)


def _cn_call(cbt):
    return pl.pallas_call(
        _cn_body,
        grid=(NC * NL,),
        in_specs=[pl.BlockSpec((1, LD, K), lambda j: (j, 0, 0))],
        out_specs=pl.BlockSpec((1, 1, K), lambda j: (j, 0, 0)),
        out_shape=jax.ShapeDtypeStruct((NC * NL, 1, K), jnp.float32),
    )(cbt)


def _codes_call(x_t, w1_bd, b1_all, fe_w2, fe_b2, pw_cat, pb_cat, cbt, cn):
    Bc = x_t.shape[0]
    return pl.pallas_call(
        _vq_codes_body,
        grid=(Bc // BT,),
        in_specs=[
            pl.BlockSpec((BT, IN), lambda i: (i, 0)),
            pl.BlockSpec((IN, NC * GH), lambda i: (0, 0)),
            pl.BlockSpec((1, NC * GH), lambda i: (0, 0)),
            pl.BlockSpec((NC, GH, GE), lambda i: (0, 0, 0)),
            pl.BlockSpec((NC, GE), lambda i: (0, 0)),
            pl.BlockSpec((NC, GE, NL * LD), lambda i: (0, 0, 0)),
            pl.BlockSpec((NC, NL * LD), lambda i: (0, 0)),
            pl.BlockSpec((NC * NL, LD, K), lambda i: (0, 0, 0)),
            pl.BlockSpec((NC * NL, 1, K), lambda i: (0, 0, 0)),
        ],
        out_specs=pl.BlockSpec((BT, 16), lambda i: (i, 0)),
        out_shape=jax.ShapeDtypeStruct((Bc, 16), jnp.int32),
    )(x_t, w1_bd, b1_all, fe_w2, fe_b2, pw_cat, pb_cat, cbt, cn)


def _decoder_call(zq3, dec_w1, dec_b1, dec_w2, dec_b2):
    Bc = zq3.shape[1]
    return pl.pallas_call(
        _decoder_body,
        grid=(Bc // DT,),
        in_specs=[
            pl.BlockSpec((NC * NL, DT, LD), lambda i: (0, i, 0)),
            pl.BlockSpec((NC * NL, LD, 256), lambda i: (0, 0, 0)),
            pl.BlockSpec((1, 256), lambda i: (0, 0)),
            pl.BlockSpec((256, IN), lambda i: (0, 0)),
            pl.BlockSpec((1, IN), lambda i: (0, 0)),
        ],
        out_specs=pl.BlockSpec((DT, IN), lambda i: (i, 0)),
        out_shape=jax.ShapeDtypeStruct((Bc, IN), jnp.float32),
    )(zq3, dec_w1, dec_b1, dec_w2, dec_b2)


NCHUNK = 2


def kernel(x, fe_w1, fe_b1, fe_w2, fe_b2, proj_w, proj_b, codebooks,
           dec_w1, dec_b1, dec_w2, dec_b2):
    B = x.shape[0]

    w1_bd = jnp.zeros((IN, NC * GH), jnp.float32)
    for c in range(NC):
        w1_bd = w1_bd.at[c * FPC:(c + 1) * FPC, c * GH:(c + 1) * GH].set(fe_w1[c])
    b1_all = fe_b1.reshape(1, NC * GH)
    pw_cat = -2.0 * jnp.transpose(proj_w.reshape(NC, NL, GE, LD),
                                  (0, 2, 1, 3)).reshape(NC, GE, NL * LD)
    pb_cat = -2.0 * proj_b.reshape(NC, NL * LD)
    cbt = jnp.transpose(codebooks, (0, 2, 1))
    cn = _cn_call(cbt)
    w1_bd = w1_bd.astype(jnp.bfloat16)
    fe_w2 = fe_w2.astype(jnp.bfloat16)
    pw_cat = pw_cat.astype(jnp.bfloat16)
    cbt = cbt.astype(jnp.bfloat16)
    cb_flat = codebooks.reshape(NC * NL * K, LD)
    db1 = dec_b1.reshape(1, 256)
    db2 = dec_b2.reshape(1, IN)
    dw1 = dec_w1.reshape(NC * NL, LD, 256).astype(jnp.bfloat16)

    sizes = [B // NCHUNK] * NCHUNK
    recons = []
    off = 0
    for t in range(NCHUNK):
        Bc = sizes[t]
        x_t = jax.lax.slice_in_dim(x, off, off + Bc, axis=0)
        off += Bc
        codes = _codes_call(x_t, w1_bd, b1_all, fe_w2, fe_b2, pw_cat,
                            pb_cat, cbt, cn)
        idx = codes[:, :NC * NL].T.reshape(Bc * NC * NL)
        zq3 = _sc_gather(cb_flat, idx).reshape(NC * NL, Bc, LD)
        recons.append(_decoder_call(zq3, dw1, db1, dec_w2, db2))
    return jnp.concatenate(recons, axis=0)


# speedup vs baseline: 1.0540x; 1.0107x over previous
import jax
import jax.numpy as jnp
from jax.experimental import pallas as pl
from jax.experimental.pallas import tpu as pltpu
from jax.experimental.pallas import tpu_sc as plsc

NC = 4
NL = 3
FPC = 32
GH = 1024
GE = 512
LD = 128
K = 2048
IN = 128

BT = 512
DT = 1024
GW = 128

_PREC = jax.lax.Precision.DEFAULT


def _cn_body(cbt_ref, cn_ref):
    cbt = cbt_ref[0]
    cn_ref[0] = jnp.sum(cbt * cbt, axis=0, keepdims=True)


def _vq_codes_body(x_ref, w1_ref, b1_ref, w2_ref, b2_ref, pw_ref, pb_ref,
                   cbt_ref, cn_ref, out_ref):
    x = x_ref[...]
    h_all = jnp.maximum(
        jnp.dot(x, w1_ref[...], preferred_element_type=jnp.float32,
                precision=_PREC) + b1_ref[...], 0.0)
    for c in range(NC):
        h = h_all[:, c * GH:(c + 1) * GH]
        emb = jnp.dot(h, w2_ref[c], preferred_element_type=jnp.float32,
                      precision=_PREC) + b2_ref[c][None, :]
        zc = jnp.dot(emb, pw_ref[c], preferred_element_type=jnp.float32,
                     precision=_PREC) + pb_ref[c][None, :]
        for l in range(NL):
            j = c * NL + l
            z2 = zc[:, l * LD:(l + 1) * LD]
            cbt = cbt_ref[j]
            scores2 = jnp.dot(z2, cbt, preferred_element_type=jnp.float32,
                              precision=_PREC)
            d = scores2 + cn_ref[j]
            m = jnp.min(d, axis=1, keepdims=True)
            iota_f = jax.lax.broadcasted_iota(
                jnp.int32, d.shape, 1).astype(jnp.float32)
            code_f = jnp.min(jnp.where(d <= m, iota_f, float(K)), axis=1)
            out_ref[:, j] = code_f.astype(jnp.int32) + j * K
    for j in range(NC * NL, 16):
        out_ref[:, j] = jnp.zeros((BT,), jnp.int32)


def _decoder_body(zq_ref, w1_ref, b1_ref, w2_ref, b2_ref, out_ref):
    acc = jnp.dot(zq_ref[0], w1_ref[0], preferred_element_type=jnp.float32,
                  precision=_PREC)
    for j in range(1, NC * NL):
        acc = acc + jnp.dot(zq_ref[j], w1_ref[j],
                            preferred_element_type=jnp.float32,
                            precision=_PREC)
    h = jnp.maximum(acc + b1_ref[...], 0.0)
    out_ref[...] = jnp.dot(h, w2_ref[...], preferred_element_type=jnp.float32,
                           precision=_PREC) + b2_ref[...]


_NW = 32
_NBUF = 4


def _sc_gather(cb_flat, idx):
    num = idx.shape[0]
    width = cb_flat.shape[1]
    per_w = num // _NW
    nblk = per_w // GW
    mesh = plsc.VectorSubcoreMesh(core_axis_name="c", subcore_axis_name="s")

    @pl.kernel(out_type=jax.ShapeDtypeStruct((num, width), cb_flat.dtype),
               mesh=mesh,
               scratch_types=([pltpu.VMEM((per_w,), jnp.int32)]
                              + [pltpu.VMEM((GW, width), cb_flat.dtype)
                                 for _ in range(_NBUF)]
                              + [pltpu.SemaphoreType.DMA] * (2 * _NBUF)))
    def gather_kernel(cb_hbm, i_hbm, o_hbm, idx_v, *bufs_and_sems):
        rows = bufs_and_sems[:_NBUF]
        sg = bufs_and_sems[_NBUF:2 * _NBUF]
        so = bufs_and_sems[2 * _NBUF:]
        wid = jax.lax.axis_index("s") * 2 + jax.lax.axis_index("c")
        base = wid * per_w
        pltpu.sync_copy(i_hbm.at[pl.ds(base, per_w)], idx_v)
        gath = [None] * _NBUF
        outc = [None] * _NBUF
        for k in range(nblk):
            b = k % _NBUF
            if k >= _NBUF:
                outc[b].wait()
            gath[b] = pltpu.async_copy(
                cb_hbm.at[idx_v.at[pl.ds(k * GW, GW)]], rows[b], sg[b])
            if k >= 2:
                p = (k - 2) % _NBUF
                gath[p].wait()
                outc[p] = pltpu.async_copy(
                    rows[p], o_hbm.at[pl.ds(base + (k - 2) * GW, GW)], so[p])
        for k in (nblk - 2, nblk - 1):
            p = k % _NBUF
            gath[p].wait()
            outc[p] = pltpu.async_copy(
                rows[p], o_hbm.at[pl.ds(base + k * GW, GW)], so[p])
        for k in range(max(0, nblk - _NBUF), nblk):
            outc[k % _NBUF].wait()

    return gather_kernel(cb_flat, idx)


def _cn_call(cbt):
    return pl.pallas_call(
        _cn_body,
        grid=(NC * NL,),
        in_specs=[pl.BlockSpec((1, LD, K), lambda j: (j, 0, 0))],
        out_specs=pl.BlockSpec((1, 1, K), lambda j: (j, 0, 0)),
        out_shape=jax.ShapeDtypeStruct((NC * NL, 1, K), jnp.float32),
    )(cbt)


def _codes_call(x_t, w1_bd, b1_all, fe_w2, fe_b2, pw_cat, pb_cat, cbt, cn):
    Bc = x_t.shape[0]
    return pl.pallas_call(
        _vq_codes_body,
        grid=(Bc // BT,),
        in_specs=[
            pl.BlockSpec((BT, IN), lambda i: (i, 0)),
            pl.BlockSpec((IN, NC * GH), lambda i: (0, 0)),
            pl.BlockSpec((1, NC * GH), lambda i: (0, 0)),
            pl.BlockSpec((NC, GH, GE), lambda i: (0, 0, 0)),
            pl.BlockSpec((NC, GE), lambda i: (0, 0)),
            pl.BlockSpec((NC, GE, NL * LD), lambda i: (0, 0, 0)),
            pl.BlockSpec((NC, NL * LD), lambda i: (0, 0)),
            pl.BlockSpec((NC * NL, LD, K), lambda i: (0, 0, 0)),
            pl.BlockSpec((NC * NL, 1, K), lambda i: (0, 0, 0)),
        ],
        out_specs=pl.BlockSpec((BT, 16), lambda i: (i, 0)),
        out_shape=jax.ShapeDtypeStruct((Bc, 16), jnp.int32),
    )(x_t, w1_bd, b1_all, fe_w2, fe_b2, pw_cat, pb_cat, cbt, cn)


def _decoder_call(zq3, dec_w1, dec_b1, dec_w2, dec_b2):
    Bc = zq3.shape[1]
    return pl.pallas_call(
        _decoder_body,
        grid=(Bc // DT,),
        in_specs=[
            pl.BlockSpec((NC * NL, DT, LD), lambda i: (0, i, 0)),
            pl.BlockSpec((NC * NL, LD, 256), lambda i: (0, 0, 0)),
            pl.BlockSpec((1, 256), lambda i: (0, 0)),
            pl.BlockSpec((256, IN), lambda i: (0, 0)),
            pl.BlockSpec((1, IN), lambda i: (0, 0)),
        ],
        out_specs=pl.BlockSpec((DT, IN), lambda i: (i, 0)),
        out_shape=jax.ShapeDtypeStruct((Bc, IN), jnp.float32),
    )(zq3, dec_w1, dec_b1, dec_w2, dec_b2)


NCHUNK = 2


def kernel(x, fe_w1, fe_b1, fe_w2, fe_b2, proj_w, proj_b, codebooks,
           dec_w1, dec_b1, dec_w2, dec_b2):
    B = x.shape[0]

    w1_bd = jnp.zeros((IN, NC * GH), jnp.float32)
    for c in range(NC):
        w1_bd = w1_bd.at[c * FPC:(c + 1) * FPC, c * GH:(c + 1) * GH].set(fe_w1[c])
    b1_all = fe_b1.reshape(1, NC * GH)
    pw_cat = -2.0 * jnp.transpose(proj_w.reshape(NC, NL, GE, LD),
                                  (0, 2, 1, 3)).reshape(NC, GE, NL * LD)
    pb_cat = -2.0 * proj_b.reshape(NC, NL * LD)
    cbt = jnp.transpose(codebooks, (0, 2, 1))
    cn = _cn_call(cbt)
    w1_bd = w1_bd.astype(jnp.bfloat16)
    fe_w2 = fe_w2.astype(jnp.bfloat16)
    pw_cat = pw_cat.astype(jnp.bfloat16)
    cbt = cbt.astype(jnp.bfloat16)
    cb_flat = codebooks.reshape(NC * NL * K, LD)
    db1 = dec_b1.reshape(1, 256)
    db2 = dec_b2.reshape(1, IN)
    dw1 = dec_w1.reshape(NC * NL, LD, 256).astype(jnp.bfloat16)

    sizes = [B // NCHUNK] * NCHUNK
    recons = []
    off = 0
    for t in range(NCHUNK):
        Bc = sizes[t]
        x_t = jax.lax.slice_in_dim(x, off, off + Bc, axis=0)
        off += Bc
        codes = _codes_call(x_t, w1_bd, b1_all, fe_w2, fe_b2, pw_cat,
                            pb_cat, cbt, cn)
        idx = codes[:, :NC * NL].T.reshape(Bc * NC * NL)
        zq3 = _sc_gather(cb_flat, idx).reshape(NC * NL, Bc, LD)
        recons.append(_decoder_call(zq3, dw1, db1, dec_w2, db2))
    return jnp.concatenate(recons, axis=0)


# speedup vs baseline: 1.0587x; 1.0045x over previous
import jax
import jax.numpy as jnp
from jax.experimental import pallas as pl
from jax.experimental.pallas import tpu as pltpu
from jax.experimental.pallas import tpu_sc as plsc

NC = 4
NL = 3
FPC = 32
GH = 1024
GE = 512
LD = 128
K = 2048
IN = 128

BT = 512
DT = 1024
GW = 128

_PREC = jax.lax.Precision.DEFAULT


def _cn_body(cbt_ref, cn_ref):
    cbt = cbt_ref[0]
    cn_ref[0] = jnp.sum(cbt * cbt, axis=0, keepdims=True)


def _vq_codes_body(x_ref, w1_ref, b1_ref, w2_ref, b2_ref, pw_ref, pb_ref,
                   cbt_ref, cn_ref, out_ref):
    x = x_ref[...]
    h_all = jnp.maximum(
        jnp.dot(x, w1_ref[...], preferred_element_type=jnp.float32,
                precision=_PREC) + b1_ref[...], 0.0)
    for c in range(NC):
        h = h_all[:, c * GH:(c + 1) * GH]
        emb = jnp.dot(h, w2_ref[c], preferred_element_type=jnp.float32,
                      precision=_PREC) + b2_ref[c][None, :]
        zc = jnp.dot(emb, pw_ref[c], preferred_element_type=jnp.float32,
                     precision=_PREC) + pb_ref[c][None, :]
        for l in range(NL):
            j = c * NL + l
            z2 = zc[:, l * LD:(l + 1) * LD]
            cbt = cbt_ref[j]
            scores2 = jnp.dot(z2, cbt, preferred_element_type=jnp.float32,
                              precision=_PREC)
            d = scores2 + cn_ref[j]
            m = jnp.min(d, axis=1, keepdims=True)
            iota_f = jax.lax.broadcasted_iota(
                jnp.int32, d.shape, 1).astype(jnp.float32)
            code_f = jnp.min(jnp.where(d <= m, iota_f, float(K)), axis=1)
            out_ref[:, j] = code_f.astype(jnp.int32) + j * K
    for j in range(NC * NL, 16):
        out_ref[:, j] = jnp.zeros((BT,), jnp.int32)


def _decoder_body(zq_ref, w1_ref, b1_ref, w2_ref, b2_ref, out_ref):
    acc = jnp.dot(zq_ref[0], w1_ref[0], preferred_element_type=jnp.float32,
                  precision=_PREC)
    for j in range(1, NC * NL):
        acc = acc + jnp.dot(zq_ref[j], w1_ref[j],
                            preferred_element_type=jnp.float32,
                            precision=_PREC)
    h = jnp.maximum(acc + b1_ref[...], 0.0)
    out_ref[...] = jnp.dot(h, w2_ref[...], preferred_element_type=jnp.float32,
                           precision=_PREC) + b2_ref[...]


_NW = 32
_NBUF = 6
_DEPTH = 3


def _sc_gather(cb_flat, idx):
    num = idx.shape[0]
    width = cb_flat.shape[1]
    per_w = num // _NW
    nblk = per_w // GW
    mesh = plsc.VectorSubcoreMesh(core_axis_name="c", subcore_axis_name="s")

    @pl.kernel(out_type=jax.ShapeDtypeStruct((num, width), cb_flat.dtype),
               mesh=mesh,
               scratch_types=([pltpu.VMEM((per_w,), jnp.int32)]
                              + [pltpu.VMEM((GW, width), cb_flat.dtype)
                                 for _ in range(_NBUF)]
                              + [pltpu.SemaphoreType.DMA] * (2 * _NBUF)))
    def gather_kernel(cb_hbm, i_hbm, o_hbm, idx_v, *bufs_and_sems):
        rows = bufs_and_sems[:_NBUF]
        sg = bufs_and_sems[_NBUF:2 * _NBUF]
        so = bufs_and_sems[2 * _NBUF:]
        wid = jax.lax.axis_index("s") * 2 + jax.lax.axis_index("c")
        base = wid * per_w
        pltpu.sync_copy(i_hbm.at[pl.ds(base, per_w)], idx_v)
        gath = [None] * _NBUF
        outc = [None] * _NBUF
        for k in range(nblk):
            b = k % _NBUF
            if k >= _NBUF:
                outc[b].wait()
            gath[b] = pltpu.async_copy(
                cb_hbm.at[idx_v.at[pl.ds(k * GW, GW)]], rows[b], sg[b])
            if k >= _DEPTH:
                p = (k - _DEPTH) % _NBUF
                gath[p].wait()
                outc[p] = pltpu.async_copy(
                    rows[p], o_hbm.at[pl.ds(base + (k - _DEPTH) * GW, GW)],
                    so[p])
        for k in range(max(0, nblk - _DEPTH), nblk):
            p = k % _NBUF
            gath[p].wait()
            outc[p] = pltpu.async_copy(
                rows[p], o_hbm.at[pl.ds(base + k * GW, GW)], so[p])
        for k in range(max(0, nblk - _NBUF), nblk):
            outc[k % _NBUF].wait()

    return gather_kernel(cb_flat, idx)


def _cn_call(cbt):
    return pl.pallas_call(
        _cn_body,
        grid=(NC * NL,),
        in_specs=[pl.BlockSpec((1, LD, K), lambda j: (j, 0, 0))],
        out_specs=pl.BlockSpec((1, 1, K), lambda j: (j, 0, 0)),
        out_shape=jax.ShapeDtypeStruct((NC * NL, 1, K), jnp.float32),
    )(cbt)


def _codes_call(x_t, w1_bd, b1_all, fe_w2, fe_b2, pw_cat, pb_cat, cbt, cn):
    Bc = x_t.shape[0]
    return pl.pallas_call(
        _vq_codes_body,
        grid=(Bc // BT,),
        in_specs=[
            pl.BlockSpec((BT, IN), lambda i: (i, 0)),
            pl.BlockSpec((IN, NC * GH), lambda i: (0, 0)),
            pl.BlockSpec((1, NC * GH), lambda i: (0, 0)),
            pl.BlockSpec((NC, GH, GE), lambda i: (0, 0, 0)),
            pl.BlockSpec((NC, GE), lambda i: (0, 0)),
            pl.BlockSpec((NC, GE, NL * LD), lambda i: (0, 0, 0)),
            pl.BlockSpec((NC, NL * LD), lambda i: (0, 0)),
            pl.BlockSpec((NC * NL, LD, K), lambda i: (0, 0, 0)),
            pl.BlockSpec((NC * NL, 1, K), lambda i: (0, 0, 0)),
        ],
        out_specs=pl.BlockSpec((BT, 16), lambda i: (i, 0)),
        out_shape=jax.ShapeDtypeStruct((Bc, 16), jnp.int32),
    )(x_t, w1_bd, b1_all, fe_w2, fe_b2, pw_cat, pb_cat, cbt, cn)


def _decoder_call(zq3, dec_w1, dec_b1, dec_w2, dec_b2):
    Bc = zq3.shape[1]
    return pl.pallas_call(
        _decoder_body,
        grid=(Bc // DT,),
        in_specs=[
            pl.BlockSpec((NC * NL, DT, LD), lambda i: (0, i, 0)),
            pl.BlockSpec((NC * NL, LD, 256), lambda i: (0, 0, 0)),
            pl.BlockSpec((1, 256), lambda i: (0, 0)),
            pl.BlockSpec((256, IN), lambda i: (0, 0)),
            pl.BlockSpec((1, IN), lambda i: (0, 0)),
        ],
        out_specs=pl.BlockSpec((DT, IN), lambda i: (i, 0)),
        out_shape=jax.ShapeDtypeStruct((Bc, IN), jnp.float32),
    )(zq3, dec_w1, dec_b1, dec_w2, dec_b2)


NCHUNK = 2


def kernel(x, fe_w1, fe_b1, fe_w2, fe_b2, proj_w, proj_b, codebooks,
           dec_w1, dec_b1, dec_w2, dec_b2):
    B = x.shape[0]

    w1_bd = jnp.zeros((IN, NC * GH), jnp.float32)
    for c in range(NC):
        w1_bd = w1_bd.at[c * FPC:(c + 1) * FPC, c * GH:(c + 1) * GH].set(fe_w1[c])
    b1_all = fe_b1.reshape(1, NC * GH)
    pw_cat = -2.0 * jnp.transpose(proj_w.reshape(NC, NL, GE, LD),
                                  (0, 2, 1, 3)).reshape(NC, GE, NL * LD)
    pb_cat = -2.0 * proj_b.reshape(NC, NL * LD)
    cbt = jnp.transpose(codebooks, (0, 2, 1))
    cn = _cn_call(cbt)
    w1_bd = w1_bd.astype(jnp.bfloat16)
    fe_w2 = fe_w2.astype(jnp.bfloat16)
    pw_cat = pw_cat.astype(jnp.bfloat16)
    cbt = cbt.astype(jnp.bfloat16)
    cb_flat = codebooks.reshape(NC * NL * K, LD)
    db1 = dec_b1.reshape(1, 256)
    db2 = dec_b2.reshape(1, IN)
    dw1 = dec_w1.reshape(NC * NL, LD, 256).astype(jnp.bfloat16)

    sizes = [B // NCHUNK] * NCHUNK
    recons = []
    off = 0
    for t in range(NCHUNK):
        Bc = sizes[t]
        x_t = jax.lax.slice_in_dim(x, off, off + Bc, axis=0)
        off += Bc
        codes = _codes_call(x_t, w1_bd, b1_all, fe_w2, fe_b2, pw_cat,
                            pb_cat, cbt, cn)
        idx = codes[:, :NC * NL].T.reshape(Bc * NC * NL)
        zq3 = _sc_gather(cb_flat, idx).reshape(NC * NL, Bc, LD)
        recons.append(_decoder_call(zq3, dw1, db1, dec_w2, db2))
    return jnp.concatenate(recons, axis=0)
